# baseline (speedup 1.0000x reference)
import functools

import jax
import jax.numpy as jnp
from jax import lax
from jax.experimental import pallas as pl
from jax.experimental.pallas import tpu as pltpu
from jax.experimental.pallas import tpu_sc as plsc

_D = 128
_LANE = 16
_NC = 2
_NS = 16
_NW = _NC * _NS
_K = 40
_WV = _D + _LANE
_D2 = 2 * _D


_PHASE = 2000


def _edge_body(e_per_tile, n_rows_per_tile,
               a_hbm, b_hbm, xl_hbm, src_hbm, dst_hbm, w_hbm, zero_hbm,
               agg_out,
               w_v, idx_s, idx_d, a_v0, b_v0, xl_v0, a_v1, b_v1, xl_v1,
               msg_v, agg_sh, sem0, sem1):
    c = lax.axis_index("c")
    s = lax.axis_index("s")
    wid = c * _NS + s

    pltpu.sync_copy(zero_hbm, agg_sh.at[pl.ds(s * n_rows_per_tile, n_rows_per_tile)])
    pltpu.sync_copy(w_hbm, w_v)
    wregs = [w_v[pl.ds(_LANE * j, _LANE)] for j in range(_D // _LANE)]
    m2b_v = w_v[pl.ds(_D, _LANE)]
    plsc.subcore_barrier()

    nblkp = _PHASE // _K

    def fetch(blk, a_v, b_v, xl_v, sem):
        ds = pl.ds(blk * _K, _K)
        pltpu.async_copy(a_hbm.at[idx_d.at[ds]], a_v, sem)
        pltpu.async_copy(b_hbm.at[idx_s.at[ds]], b_v, sem)
        pltpu.async_copy(xl_hbm.at[idx_s.at[ds]], xl_v, sem)

    def drain(blk, a_v, b_v, xl_v, sem):
        ds = pl.ds(blk * _K, _K)
        pltpu.make_async_copy(a_hbm.at[idx_d.at[ds]], a_v, sem).wait()
        pltpu.make_async_copy(b_hbm.at[idx_s.at[ds]], b_v, sem).wait()
        pltpu.make_async_copy(xl_hbm.at[idx_s.at[ds]], xl_v, sem).wait()

    def gate(off, a_v, b_v, xl_v):
        @plsc.parallel_loop(0, _K, step=1, unroll=4)
        def edge(e):
            acc = None
            for j in range(_D // _LANE):
                av = a_v[e, pl.ds(_LANE * j, _LANE)]
                bv = b_v[e, pl.ds(_LANE * j, _LANE)]
                h = jnp.maximum(av + bv, 0.0)
                t = h * wregs[j]
                acc = t if acc is None else acc + t
            z = jnp.full((_LANE,), jnp.sum(acc), jnp.float32) + m2b_v
            sig = 1.0 / (1.0 + jnp.exp(-z))
            for j in range(_D // _LANE):
                msg_v[off + e, pl.ds(_LANE * j, _LANE)] = (
                    xl_v[e, pl.ds(_LANE * j, _LANE)] * sig)

    def phase(p, carry):
        base = wid * e_per_tile + p * _PHASE
        pltpu.sync_copy(src_hbm.at[pl.ds(base, _PHASE)], idx_s)
        pltpu.sync_copy(dst_hbm.at[pl.ds(base, _PHASE)], idx_d)
        fetch(0, a_v0, b_v0, xl_v0, sem0)

        def pair(i, carry2):
            blk_a = 2 * i
            fetch(blk_a + 1, a_v1, b_v1, xl_v1, sem1)
            drain(blk_a, a_v0, b_v0, xl_v0, sem0)
            gate(0, a_v0, b_v0, xl_v0)
            fetch(jnp.minimum(blk_a + 2, nblkp - 2), a_v0, b_v0, xl_v0, sem0)
            drain(blk_a + 1, a_v1, b_v1, xl_v1, sem1)
            gate(_K, a_v1, b_v1, xl_v1)
            pltpu.sync_copy(msg_v,
                            agg_sh.at[idx_d.at[pl.ds(blk_a * _K, 2 * _K)]],
                            add=True)
            return carry2

        lax.fori_loop(0, nblkp // 2, pair, 0)
        drain(nblkp - 2, a_v0, b_v0, xl_v0, sem0)
        return carry

    lax.fori_loop(0, e_per_tile // _PHASE, phase, 0)
    plsc.subcore_barrier()
    pltpu.sync_copy(agg_sh.at[pl.ds(s * n_rows_per_tile, n_rows_per_tile)],
                    agg_out.at[c, pl.ds(s * n_rows_per_tile, n_rows_per_tile)])


@functools.cache
def _make_edge_call(n_nodes, n_edges):
    assert n_edges % _NW == 0, n_edges
    e_per_tile = n_edges // _NW
    assert e_per_tile % _PHASE == 0 and _PHASE % (2 * _K) == 0
    n_pad = -(-n_nodes // (8 * _NS)) * (8 * _NS)
    rows = n_pad // _NS
    mesh = plsc.VectorSubcoreMesh(core_axis_name="c", subcore_axis_name="s")
    return pl.kernel(
        functools.partial(_edge_body, e_per_tile, rows),
        out_type=jax.ShapeDtypeStruct((_NC, n_pad, _D), jnp.float32),
        mesh=mesh,
        compiler_params=pltpu.CompilerParams(needs_layout_passes=False),
        scratch_types=[
            pltpu.VMEM((_WV,), jnp.float32),
            pltpu.VMEM((_PHASE,), jnp.int32),
            pltpu.VMEM((_PHASE,), jnp.int32),
            pltpu.VMEM((_K, _D), jnp.float32),
            pltpu.VMEM((_K, _D), jnp.float32),
            pltpu.VMEM((_K, _D), jnp.float32),
            pltpu.VMEM((_K, _D), jnp.float32),
            pltpu.VMEM((_K, _D), jnp.float32),
            pltpu.VMEM((_K, _D), jnp.float32),
            pltpu.VMEM((2 * _K, _D), jnp.float32),
            pltpu.VMEM_SHARED((n_pad, _D), jnp.float32),
            pltpu.SemaphoreType.DMA,
            pltpu.SemaphoreType.DMA,
        ],
    )


def _count_body(e_per_tile, n_rows_per_tile,
                dst_hbm, ones_hbm, zero_hbm, cnt_out,
                idx_d, ones_v, cnt_sh):
    c = lax.axis_index("c")
    s = lax.axis_index("s")
    wid = c * _NS + s

    pltpu.sync_copy(zero_hbm, cnt_sh.at[pl.ds(s * n_rows_per_tile, n_rows_per_tile)])
    pltpu.sync_copy(ones_hbm, ones_v)
    pltpu.sync_copy(dst_hbm.at[pl.ds(wid * e_per_tile, e_per_tile)], idx_d)
    plsc.subcore_barrier()

    nblk = e_per_tile // _K

    def block(blk, carry):
        pltpu.sync_copy(ones_v, cnt_sh.at[idx_d.at[pl.ds(blk * _K, _K)]], add=True)
        return carry

    lax.fori_loop(0, nblk, block, 0)
    plsc.subcore_barrier()
    pltpu.sync_copy(cnt_sh.at[pl.ds(s * n_rows_per_tile, n_rows_per_tile)],
                    cnt_out.at[c, pl.ds(s * n_rows_per_tile, n_rows_per_tile)])


@functools.cache
def _make_count_call(n_nodes, n_edges):
    e_per_tile = n_edges // _NW
    n_pad = -(-n_nodes // (8 * _NS)) * (8 * _NS)
    rows = n_pad // _NS
    mesh = plsc.VectorSubcoreMesh(core_axis_name="c", subcore_axis_name="s")
    return pl.kernel(
        functools.partial(_count_body, e_per_tile, rows),
        out_type=jax.ShapeDtypeStruct((_NC, n_pad, _D), jnp.float32),
        mesh=mesh,
        compiler_params=pltpu.CompilerParams(needs_layout_passes=False),
        scratch_types=[
            pltpu.VMEM((e_per_tile,), jnp.int32),
            pltpu.VMEM((_K, _D), jnp.float32),
            pltpu.VMEM_SHARED((n_pad, _D), jnp.float32),
        ],
    )


_BLK = 2000


def _pre_body(x_ref, wtop_ref, wbot_ref, m1b_ref, linw_ref, m2w_ref, m2b_ref,
              a_ref, b_ref, xl_ref, self_ref):
    x = x_ref[...]
    a = jnp.dot(x, wtop_ref[...], preferred_element_type=jnp.float32) + m1b_ref[...]
    b = jnp.dot(x, wbot_ref[...], preferred_element_type=jnp.float32)
    xl = jnp.dot(x, linw_ref[...], preferred_element_type=jnp.float32)
    h = jnp.maximum(a + b, 0.0)
    z = jnp.sum(h * m2w_ref[...], axis=1, keepdims=True) + m2b_ref[...][:, :1]
    sig = 1.0 / (1.0 + jnp.exp(-z))
    a_ref[...] = a
    b_ref[...] = b
    xl_ref[...] = xl
    self_ref[...] = sig * xl


@functools.cache
def _make_pre_call(n_nodes):
    assert n_nodes % _BLK == 0
    grid = (n_nodes // _BLK,)
    full = lambda i: (0, 0)
    blk = lambda i: (i, 0)
    node_spec = pl.BlockSpec((_BLK, _D), blk)
    return pl.pallas_call(
        _pre_body,
        grid=grid,
        in_specs=[
            node_spec,
            pl.BlockSpec((_D, _D), full),
            pl.BlockSpec((_D, _D), full),
            pl.BlockSpec((1, _D), full),
            pl.BlockSpec((_D, _D), full),
            pl.BlockSpec((1, _D), full),
            pl.BlockSpec((1, _D), full),
        ],
        out_specs=[node_spec] * 4,
        out_shape=[jax.ShapeDtypeStruct((n_nodes, _D), jnp.float32)] * 4,
    )


def _combine_body(do_relu, agg_ref, cnt_ref, self_ref, linb_ref, out_ref):
    ssum = agg_ref[0] + agg_ref[1]
    csum = cnt_ref[0] + cnt_ref[1]
    tot = ssum + self_ref[...]
    cnt = csum[:, :1] + 1.0
    out = tot / cnt + linb_ref[...]
    if do_relu:
        out = jnp.maximum(out, 0.0)
    out_ref[...] = out


@functools.cache
def _make_combine_call(n_nodes, do_relu):
    grid = (n_nodes // _BLK,)
    return pl.pallas_call(
        functools.partial(_combine_body, do_relu),
        grid=grid,
        in_specs=[
            pl.BlockSpec((_NC, _BLK, _D), lambda i: (0, i, 0)),
            pl.BlockSpec((_NC, _BLK, _D), lambda i: (0, i, 0)),
            pl.BlockSpec((_BLK, _D), lambda i: (i, 0)),
            pl.BlockSpec((1, _D), lambda i: (0, 0)),
        ],
        out_specs=pl.BlockSpec((_BLK, _D), lambda i: (i, 0)),
        out_shape=jax.ShapeDtypeStruct((n_nodes, _D), jnp.float32),
    )


def _layer(xn, src, dst, cnt, m1W, m1b, m2W, m2b, linW, linb, do_relu):
    n = xn.shape[0]
    e = src.shape[0]
    pre = _make_pre_call(n)
    a, b, xl, selfmsg = pre(
        xn, m1W[:_D], m1W[_D:], m1b.reshape(1, _D), linW,
        m2W.reshape(1, _D), jnp.broadcast_to(m2b.reshape(1, 1), (1, _D)))
    wparams = jnp.concatenate([m2W[:, 0], jnp.broadcast_to(m2b, (_LANE,))])
    n_pad = -(-n // (8 * _NS)) * (8 * _NS)
    zeros = jnp.zeros((n_pad // _NS, _D), jnp.float32)
    agg = _make_edge_call(n, e)(a, b, xl, src, dst, wparams, zeros)
    return _make_combine_call(n, do_relu)(agg, cnt, selfmsg, linb.reshape(1, _D))


def kernel(x, edge_index, Wl1, bl1, c1_m1W, c1_m1b, c1_m2W, c1_m2b, c1_linW,
           c1_linb, c2_m1W, c2_m1b, c2_m2W, c2_m2b, c2_linW, c2_linb):
    src = edge_index[0]
    dst = edge_index[1]
    n = x.shape[0]
    e = src.shape[0]
    n_pad = -(-n // (8 * _NS)) * (8 * _NS)
    onehot = jnp.zeros((_K, _D), jnp.float32).at[:, 0].set(1.0)
    zeros = jnp.zeros((n_pad // _NS, _D), jnp.float32)
    cnt = _make_count_call(n, e)(dst, onehot, zeros)
    x32 = _layer(x, src, dst, cnt, c1_m1W, c1_m1b, c1_m2W, c1_m2b, c1_linW,
                 c1_linb, do_relu=True)
    h2 = _layer(x32, src, dst, cnt, c2_m1W, c2_m1b, c2_m2W, c2_m2b, c2_linW,
                c2_linb, do_relu=False)
    return (x32, h2)


# speedup vs baseline: 1.1256x; 1.1256x over previous
import functools

import jax
import jax.numpy as jnp
from jax import lax
from jax.experimental import pallas as pl
from jax.experimental.pallas import tpu as pltpu
from jax.experimental.pallas import tpu_sc as plsc

_D = 128
_LANE = 16
_NC = 2
_NS = 16
_NW = _NC * _NS
_K = 40
_WV = _D + _LANE
_D2 = 2 * _D


_PHASE = 2000


def _edge_body(e_per_tile, n_rows_per_tile,
               a_hbm, b_hbm, xl_hbm, src_hbm, dst_hbm, w_hbm, zero_hbm,
               agg_out,
               w_v, idx_s, idx_d, a_v0, b_v0, xl_v0, a_v1, b_v1, xl_v1,
               msg_v0, msg_v1, agg_sh, sem0, sem1, sems0, sems1):
    c = lax.axis_index("c")
    s = lax.axis_index("s")
    wid = c * _NS + s

    pltpu.sync_copy(zero_hbm, agg_sh.at[pl.ds(s * n_rows_per_tile, n_rows_per_tile)])
    pltpu.sync_copy(w_hbm, w_v)
    wregs = [w_v[pl.ds(_LANE * j, _LANE)] for j in range(_D // _LANE)]
    m2b_v = w_v[pl.ds(_D, _LANE)]
    plsc.subcore_barrier()

    nblkp = _PHASE // _K

    def fetch(blk, a_v, b_v, xl_v, sem):
        ds = pl.ds(blk * _K, _K)
        pltpu.async_copy(a_hbm.at[idx_d.at[ds]], a_v, sem)
        pltpu.async_copy(b_hbm.at[idx_s.at[ds]], b_v, sem)
        pltpu.async_copy(xl_hbm.at[idx_s.at[ds]], xl_v, sem)

    def drain(blk, a_v, b_v, xl_v, sem):
        ds = pl.ds(blk * _K, _K)
        pltpu.make_async_copy(a_hbm.at[idx_d.at[ds]], a_v, sem).wait()
        pltpu.make_async_copy(b_hbm.at[idx_s.at[ds]], b_v, sem).wait()
        pltpu.make_async_copy(xl_hbm.at[idx_s.at[ds]], xl_v, sem).wait()

    def gate(msg_v, a_v, b_v, xl_v):
        def edge(e, carry2):
            acc = None
            for j in range(_D // _LANE):
                av = a_v[e, pl.ds(_LANE * j, _LANE)]
                bv = b_v[e, pl.ds(_LANE * j, _LANE)]
                h = jnp.maximum(av + bv, 0.0)
                t = h * wregs[j]
                acc = t if acc is None else acc + t
            z = jnp.full((_LANE,), jnp.sum(acc), jnp.float32) + m2b_v
            sig = 1.0 / (1.0 + jnp.exp(-z))
            for j in range(_D // _LANE):
                msg_v[e, pl.ds(_LANE * j, _LANE)] = (
                    xl_v[e, pl.ds(_LANE * j, _LANE)] * sig)
            return carry2

        lax.fori_loop(0, _K, edge, 0)

    def scat(blk, msg_v, sem):
        pltpu.async_copy(msg_v, agg_sh.at[idx_d.at[pl.ds(blk * _K, _K)]],
                         sem, add=True)

    def scat_drain(blk, msg_v, sem):
        pltpu.make_async_copy(msg_v, agg_sh.at[idx_d.at[pl.ds(blk * _K, _K)]],
                              sem).wait()

    def phase(p, carry):
        base = wid * e_per_tile + p * _PHASE
        pltpu.sync_copy(src_hbm.at[pl.ds(base, _PHASE)], idx_s)
        pltpu.sync_copy(dst_hbm.at[pl.ds(base, _PHASE)], idx_d)
        fetch(0, a_v0, b_v0, xl_v0, sem0)

        def pair(i, carry2):
            blk_a = 2 * i
            fetch(blk_a + 1, a_v1, b_v1, xl_v1, sem1)
            drain(blk_a, a_v0, b_v0, xl_v0, sem0)

            @pl.when(i > 0)
            def _():
                scat_drain(blk_a - 2, msg_v0, sems0)

            gate(msg_v0, a_v0, b_v0, xl_v0)
            scat(blk_a, msg_v0, sems0)
            fetch(jnp.minimum(blk_a + 2, nblkp - 2), a_v0, b_v0, xl_v0, sem0)
            drain(blk_a + 1, a_v1, b_v1, xl_v1, sem1)

            @pl.when(i > 0)
            def _():
                scat_drain(blk_a - 1, msg_v1, sems1)

            gate(msg_v1, a_v1, b_v1, xl_v1)
            scat(blk_a + 1, msg_v1, sems1)
            return carry2

        lax.fori_loop(0, nblkp // 2, pair, 0)
        drain(nblkp - 2, a_v0, b_v0, xl_v0, sem0)
        scat_drain(nblkp - 2, msg_v0, sems0)
        scat_drain(nblkp - 1, msg_v1, sems1)
        return carry

    lax.fori_loop(0, e_per_tile // _PHASE, phase, 0)
    plsc.subcore_barrier()
    pltpu.sync_copy(agg_sh.at[pl.ds(s * n_rows_per_tile, n_rows_per_tile)],
                    agg_out.at[c, pl.ds(s * n_rows_per_tile, n_rows_per_tile)])


@functools.cache
def _make_edge_call(n_nodes, n_edges):
    assert n_edges % _NW == 0, n_edges
    e_per_tile = n_edges // _NW
    assert e_per_tile % _PHASE == 0 and _PHASE % (2 * _K) == 0
    n_pad = -(-n_nodes // (8 * _NS)) * (8 * _NS)
    rows = n_pad // _NS
    mesh = plsc.VectorSubcoreMesh(core_axis_name="c", subcore_axis_name="s")
    return pl.kernel(
        functools.partial(_edge_body, e_per_tile, rows),
        out_type=jax.ShapeDtypeStruct((_NC, n_pad, _D), jnp.float32),
        mesh=mesh,
        compiler_params=pltpu.CompilerParams(needs_layout_passes=False),
        scratch_types=[
            pltpu.VMEM((_WV,), jnp.float32),
            pltpu.VMEM((_PHASE,), jnp.int32),
            pltpu.VMEM((_PHASE,), jnp.int32),
            pltpu.VMEM((_K, _D), jnp.float32),
            pltpu.VMEM((_K, _D), jnp.float32),
            pltpu.VMEM((_K, _D), jnp.float32),
            pltpu.VMEM((_K, _D), jnp.float32),
            pltpu.VMEM((_K, _D), jnp.float32),
            pltpu.VMEM((_K, _D), jnp.float32),
            pltpu.VMEM((_K, _D), jnp.float32),
            pltpu.VMEM((_K, _D), jnp.float32),
            pltpu.VMEM_SHARED((n_pad, _D), jnp.float32),
            pltpu.SemaphoreType.DMA,
            pltpu.SemaphoreType.DMA,
            pltpu.SemaphoreType.DMA,
            pltpu.SemaphoreType.DMA,
        ],
    )


def _count_body(e_per_tile, n_rows_per_tile,
                dst_hbm, ones_hbm, zero_hbm, cnt_out,
                idx_d, ones_v, cnt_sh):
    c = lax.axis_index("c")
    s = lax.axis_index("s")
    wid = c * _NS + s

    pltpu.sync_copy(zero_hbm, cnt_sh.at[pl.ds(s * n_rows_per_tile, n_rows_per_tile)])
    pltpu.sync_copy(ones_hbm, ones_v)
    pltpu.sync_copy(dst_hbm.at[pl.ds(wid * e_per_tile, e_per_tile)], idx_d)
    plsc.subcore_barrier()

    nblk = e_per_tile // _K

    def block(blk, carry):
        pltpu.sync_copy(ones_v, cnt_sh.at[idx_d.at[pl.ds(blk * _K, _K)]], add=True)
        return carry

    lax.fori_loop(0, nblk, block, 0)
    plsc.subcore_barrier()
    pltpu.sync_copy(cnt_sh.at[pl.ds(s * n_rows_per_tile, n_rows_per_tile)],
                    cnt_out.at[c, pl.ds(s * n_rows_per_tile, n_rows_per_tile)])


@functools.cache
def _make_count_call(n_nodes, n_edges):
    e_per_tile = n_edges // _NW
    n_pad = -(-n_nodes // (8 * _NS)) * (8 * _NS)
    rows = n_pad // _NS
    mesh = plsc.VectorSubcoreMesh(core_axis_name="c", subcore_axis_name="s")
    return pl.kernel(
        functools.partial(_count_body, e_per_tile, rows),
        out_type=jax.ShapeDtypeStruct((_NC, n_pad, _D), jnp.float32),
        mesh=mesh,
        compiler_params=pltpu.CompilerParams(needs_layout_passes=False),
        scratch_types=[
            pltpu.VMEM((e_per_tile,), jnp.int32),
            pltpu.VMEM((_K, _D), jnp.float32),
            pltpu.VMEM_SHARED((n_pad, _D), jnp.float32),
        ],
    )


_BLK = 2000


def _pre_body(x_ref, wtop_ref, wbot_ref, m1b_ref, linw_ref, m2w_ref, m2b_ref,
              a_ref, b_ref, xl_ref, self_ref):
    x = x_ref[...]
    a = jnp.dot(x, wtop_ref[...], preferred_element_type=jnp.float32) + m1b_ref[...]
    b = jnp.dot(x, wbot_ref[...], preferred_element_type=jnp.float32)
    xl = jnp.dot(x, linw_ref[...], preferred_element_type=jnp.float32)
    h = jnp.maximum(a + b, 0.0)
    z = jnp.sum(h * m2w_ref[...], axis=1, keepdims=True) + m2b_ref[...][:, :1]
    sig = 1.0 / (1.0 + jnp.exp(-z))
    a_ref[...] = a
    b_ref[...] = b
    xl_ref[...] = xl
    self_ref[...] = sig * xl


@functools.cache
def _make_pre_call(n_nodes):
    assert n_nodes % _BLK == 0
    grid = (n_nodes // _BLK,)
    full = lambda i: (0, 0)
    blk = lambda i: (i, 0)
    node_spec = pl.BlockSpec((_BLK, _D), blk)
    return pl.pallas_call(
        _pre_body,
        grid=grid,
        in_specs=[
            node_spec,
            pl.BlockSpec((_D, _D), full),
            pl.BlockSpec((_D, _D), full),
            pl.BlockSpec((1, _D), full),
            pl.BlockSpec((_D, _D), full),
            pl.BlockSpec((1, _D), full),
            pl.BlockSpec((1, _D), full),
        ],
        out_specs=[node_spec] * 4,
        out_shape=[jax.ShapeDtypeStruct((n_nodes, _D), jnp.float32)] * 4,
    )


def _combine_body(do_relu, agg_ref, cnt_ref, self_ref, linb_ref, out_ref):
    ssum = agg_ref[0] + agg_ref[1]
    csum = cnt_ref[0] + cnt_ref[1]
    tot = ssum + self_ref[...]
    cnt = csum[:, :1] + 1.0
    out = tot / cnt + linb_ref[...]
    if do_relu:
        out = jnp.maximum(out, 0.0)
    out_ref[...] = out


@functools.cache
def _make_combine_call(n_nodes, do_relu):
    grid = (n_nodes // _BLK,)
    return pl.pallas_call(
        functools.partial(_combine_body, do_relu),
        grid=grid,
        in_specs=[
            pl.BlockSpec((_NC, _BLK, _D), lambda i: (0, i, 0)),
            pl.BlockSpec((_NC, _BLK, _D), lambda i: (0, i, 0)),
            pl.BlockSpec((_BLK, _D), lambda i: (i, 0)),
            pl.BlockSpec((1, _D), lambda i: (0, 0)),
        ],
        out_specs=pl.BlockSpec((_BLK, _D), lambda i: (i, 0)),
        out_shape=jax.ShapeDtypeStruct((n_nodes, _D), jnp.float32),
    )


def _layer(xn, src, dst, cnt, m1W, m1b, m2W, m2b, linW, linb, do_relu):
    n = xn.shape[0]
    e = src.shape[0]
    pre = _make_pre_call(n)
    a, b, xl, selfmsg = pre(
        xn, m1W[:_D], m1W[_D:], m1b.reshape(1, _D), linW,
        m2W.reshape(1, _D), jnp.broadcast_to(m2b.reshape(1, 1), (1, _D)))
    wparams = jnp.concatenate([m2W[:, 0], jnp.broadcast_to(m2b, (_LANE,))])
    n_pad = -(-n // (8 * _NS)) * (8 * _NS)
    zeros = jnp.zeros((n_pad // _NS, _D), jnp.float32)
    agg = _make_edge_call(n, e)(a, b, xl, src, dst, wparams, zeros)
    return _make_combine_call(n, do_relu)(agg, cnt, selfmsg, linb.reshape(1, _D))


def kernel(x, edge_index, Wl1, bl1, c1_m1W, c1_m1b, c1_m2W, c1_m2b, c1_linW,
           c1_linb, c2_m1W, c2_m1b, c2_m2W, c2_m2b, c2_linW, c2_linb):
    src = edge_index[0]
    dst = edge_index[1]
    n = x.shape[0]
    e = src.shape[0]
    n_pad = -(-n // (8 * _NS)) * (8 * _NS)
    onehot = jnp.zeros((_K, _D), jnp.float32).at[:, 0].set(1.0)
    zeros = jnp.zeros((n_pad // _NS, _D), jnp.float32)
    cnt = _make_count_call(n, e)(dst, onehot, zeros)
    x32 = _layer(x, src, dst, cnt, c1_m1W, c1_m1b, c1_m2W, c1_m2b, c1_linW,
                 c1_linb, do_relu=True)
    h2 = _layer(x32, src, dst, cnt, c2_m1W, c2_m1b, c2_m2W, c2_m2b, c2_linW,
                c2_linb, do_relu=False)
    return (x32, h2)


# speedup vs baseline: 1.2202x; 1.0840x over previous
import functools

import jax
import jax.numpy as jnp
from jax import lax
from jax.experimental import pallas as pl
from jax.experimental.pallas import tpu as pltpu
from jax.experimental.pallas import tpu_sc as plsc

_D = 128
_LANE = 16
_NC = 2
_NS = 16
_NW = _NC * _NS
_K = 40
_WV = _D + _LANE


_CL = 32


def _cnt_rows(n_pad):
    r = -(-n_pad // _CL)
    return -(-r // 8) * 8


_PHASE = 2000


def _edge_body(e_per_tile, n_rows_per_tile, n_pad,
               a_hbm, b_hbm, xl_hbm, src_hbm, dst_hbm, w_hbm,
               zero_hbm, zero2_hbm, ones_hbm, agg_out, cnt_out,
               w_v, idx_s, idx_d, ones_v, a_v0, b_v0, xl_v0, a_v1, b_v1,
               xl_v1, msg_v0, msg_v1, agg_sh, cnt_sh,
               sem0, sem1, sems0, sems1, semc0, semc1):
    c = lax.axis_index("c")
    s = lax.axis_index("s")
    wid = c * _NS + s

    pltpu.sync_copy(zero_hbm, agg_sh.at[pl.ds(s * n_rows_per_tile, n_rows_per_tile)])

    n_cslice = n_pad // _NS
    pltpu.sync_copy(zero2_hbm.at[pl.ds(s * n_cslice, n_cslice)],
                    idx_s.at[pl.ds(0, n_cslice)])
    pltpu.sync_copy(idx_s.at[pl.ds(0, n_cslice)],
                    cnt_sh.at[pl.ds(s * n_cslice, n_cslice)])
    pltpu.sync_copy(w_hbm, w_v)
    pltpu.sync_copy(ones_hbm, ones_v)
    wregs = [w_v[pl.ds(_LANE * j, _LANE)] for j in range(_D // _LANE)]
    m2b_v = w_v[pl.ds(_D, _LANE)]
    plsc.subcore_barrier()

    nblkp = _PHASE // _K

    def fetch(blk, a_v, b_v, xl_v, sem):
        ds = pl.ds(blk * _K, _K)
        pltpu.async_copy(a_hbm.at[idx_d.at[ds]], a_v, sem)
        pltpu.async_copy(b_hbm.at[idx_s.at[ds]], b_v, sem)
        pltpu.async_copy(xl_hbm.at[idx_s.at[ds]], xl_v, sem)

    def drain(blk, a_v, b_v, xl_v, sem):
        ds = pl.ds(blk * _K, _K)
        pltpu.make_async_copy(a_hbm.at[idx_d.at[ds]], a_v, sem).wait()
        pltpu.make_async_copy(b_hbm.at[idx_s.at[ds]], b_v, sem).wait()
        pltpu.make_async_copy(xl_hbm.at[idx_s.at[ds]], xl_v, sem).wait()

    def gate(msg_v, a_v, b_v, xl_v):
        def edge(e, carry2):
            acc = None
            for j in range(_D // _LANE):
                av = a_v[e, pl.ds(_LANE * j, _LANE)]
                bv = b_v[e, pl.ds(_LANE * j, _LANE)]
                h = jnp.maximum(av + bv, 0.0)
                t = h * wregs[j]
                acc = t if acc is None else acc + t
            z = jnp.full((_LANE,), jnp.sum(acc), jnp.float32) + m2b_v
            sig = 1.0 / (1.0 + jnp.exp(-z))
            for j in range(_D // _LANE):
                msg_v[e, pl.ds(_LANE * j, _LANE)] = (
                    xl_v[e, pl.ds(_LANE * j, _LANE)] * sig)
            return carry2

        lax.fori_loop(0, _K, edge, 0)

    def scat(blk, msg_v, sem, semc):
        ds = pl.ds(blk * _K, _K)
        pltpu.async_copy(msg_v, agg_sh.at[idx_d.at[ds]], sem, add=True)
        pltpu.async_copy(ones_v, cnt_sh.at[idx_d.at[ds]], semc, add=True)

    def scat_drain(blk, msg_v, sem, semc):
        ds = pl.ds(blk * _K, _K)
        pltpu.make_async_copy(msg_v, agg_sh.at[idx_d.at[ds]], sem).wait()
        pltpu.make_async_copy(ones_v, cnt_sh.at[idx_d.at[ds]], semc).wait()

    def phase(p, carry):
        base = wid * e_per_tile + p * _PHASE
        pltpu.sync_copy(src_hbm.at[pl.ds(base, _PHASE)], idx_s)
        pltpu.sync_copy(dst_hbm.at[pl.ds(base, _PHASE)], idx_d)
        fetch(0, a_v0, b_v0, xl_v0, sem0)

        def pair(i, carry2):
            blk_a = 2 * i
            fetch(blk_a + 1, a_v1, b_v1, xl_v1, sem1)
            drain(blk_a, a_v0, b_v0, xl_v0, sem0)

            @pl.when(i > 0)
            def _():
                scat_drain(blk_a - 2, msg_v0, sems0, semc0)

            gate(msg_v0, a_v0, b_v0, xl_v0)
            scat(blk_a, msg_v0, sems0, semc0)
            fetch(jnp.minimum(blk_a + 2, nblkp - 2), a_v0, b_v0, xl_v0, sem0)
            drain(blk_a + 1, a_v1, b_v1, xl_v1, sem1)

            @pl.when(i > 0)
            def _():
                scat_drain(blk_a - 1, msg_v1, sems1, semc1)

            gate(msg_v1, a_v1, b_v1, xl_v1)
            scat(blk_a + 1, msg_v1, sems1, semc1)
            return carry2

        lax.fori_loop(0, nblkp // 2, pair, 0)
        drain(nblkp - 2, a_v0, b_v0, xl_v0, sem0)
        scat_drain(nblkp - 2, msg_v0, sems0, semc0)
        scat_drain(nblkp - 1, msg_v1, sems1, semc1)
        return carry

    lax.fori_loop(0, e_per_tile // _PHASE, phase, 0)
    plsc.subcore_barrier()
    pltpu.sync_copy(agg_sh.at[pl.ds(s * n_rows_per_tile, n_rows_per_tile)],
                    agg_out.at[c, pl.ds(s * n_rows_per_tile, n_rows_per_tile)])

    pltpu.sync_copy(cnt_sh.at[pl.ds(s * n_cslice, n_cslice)],
                    idx_s.at[pl.ds(0, n_cslice)])
    pltpu.sync_copy(idx_s.at[pl.ds(0, n_cslice)],
                    cnt_out.at[pl.ds(c * n_pad + s * n_cslice, n_cslice)])


@functools.cache
def _make_edge_call(n_nodes, n_edges):
    assert n_edges % _NW == 0, n_edges
    e_per_tile = n_edges // _NW
    assert e_per_tile % _PHASE == 0 and _PHASE % (2 * _K) == 0
    n_pad = -(-n_nodes // (8 * _NS)) * (8 * _NS)
    rows = n_pad // _NS
    mesh = plsc.VectorSubcoreMesh(core_axis_name="c", subcore_axis_name="s")
    out_type = [
        jax.ShapeDtypeStruct((_NC, n_pad, _D), jnp.float32),
        jax.ShapeDtypeStruct((_NC * n_pad,), jnp.int32),
    ]
    scratch = [
        pltpu.VMEM((_WV,), jnp.float32),
        pltpu.VMEM((_PHASE,), jnp.int32),
        pltpu.VMEM((_PHASE,), jnp.int32),
        pltpu.VMEM((_K,), jnp.int32),
        pltpu.VMEM((_K, _D), jnp.float32),
        pltpu.VMEM((_K, _D), jnp.float32),
        pltpu.VMEM((_K, _D), jnp.float32),
        pltpu.VMEM((_K, _D), jnp.float32),
        pltpu.VMEM((_K, _D), jnp.float32),
        pltpu.VMEM((_K, _D), jnp.float32),
        pltpu.VMEM((_K, _D), jnp.float32),
        pltpu.VMEM((_K, _D), jnp.float32),
        pltpu.VMEM_SHARED((n_pad, _D), jnp.float32),
        pltpu.VMEM_SHARED((n_pad,), jnp.int32),
    ]
    scratch += [pltpu.SemaphoreType.DMA] * 6
    return pl.kernel(
        functools.partial(_edge_body, e_per_tile, rows, n_pad),
        out_type=out_type,
        mesh=mesh,
        compiler_params=pltpu.CompilerParams(needs_layout_passes=False),
        scratch_types=scratch,
    )


_BLK = 2000


def _pre_body(x_ref, wtop_ref, wbot_ref, m1b_ref, linw_ref, m2w_ref, m2b_ref,
              a_ref, b_ref, xl_ref, self_ref):
    x = x_ref[...]
    a = jnp.dot(x, wtop_ref[...], preferred_element_type=jnp.float32) + m1b_ref[...]
    b = jnp.dot(x, wbot_ref[...], preferred_element_type=jnp.float32)
    xl = jnp.dot(x, linw_ref[...], preferred_element_type=jnp.float32)
    h = jnp.maximum(a + b, 0.0)
    z = jnp.sum(h * m2w_ref[...], axis=1, keepdims=True) + m2b_ref[...][:, :1]
    sig = 1.0 / (1.0 + jnp.exp(-z))
    a_ref[...] = a
    b_ref[...] = b
    xl_ref[...] = xl
    self_ref[...] = sig * xl


@functools.cache
def _make_pre_call(n_nodes):
    assert n_nodes % _BLK == 0
    grid = (n_nodes // _BLK,)
    full = lambda i: (0, 0)
    blk = lambda i: (i, 0)
    node_spec = pl.BlockSpec((_BLK, _D), blk)
    return pl.pallas_call(
        _pre_body,
        grid=grid,
        in_specs=[
            node_spec,
            pl.BlockSpec((_D, _D), full),
            pl.BlockSpec((_D, _D), full),
            pl.BlockSpec((1, _D), full),
            pl.BlockSpec((_D, _D), full),
            pl.BlockSpec((1, _D), full),
            pl.BlockSpec((1, _D), full),
        ],
        out_specs=[node_spec] * 4,
        out_shape=[jax.ShapeDtypeStruct((n_nodes, _D), jnp.float32)] * 4,
    )


def _combine_body(do_relu, agg_ref, cnt_ref, self_ref, linb_ref, out_ref):
    ssum = agg_ref[0] + agg_ref[1]
    tot = ssum + self_ref[...]
    cnt = cnt_ref[...] + 1.0
    out = tot / cnt + linb_ref[...]
    if do_relu:
        out = jnp.maximum(out, 0.0)
    out_ref[...] = out


@functools.cache
def _make_combine_call(n_nodes, do_relu):
    grid = (n_nodes // _BLK,)
    return pl.pallas_call(
        functools.partial(_combine_body, do_relu),
        grid=grid,
        in_specs=[
            pl.BlockSpec((_NC, _BLK, _D), lambda i: (0, i, 0)),
            pl.BlockSpec((_BLK, 1), lambda i: (i, 0)),
            pl.BlockSpec((_BLK, _D), lambda i: (i, 0)),
            pl.BlockSpec((1, _D), lambda i: (0, 0)),
        ],
        out_specs=pl.BlockSpec((_BLK, _D), lambda i: (i, 0)),
        out_shape=jax.ShapeDtypeStruct((n_nodes, _D), jnp.float32),
    )


def _layer(xn, src, dst, cnt_vec, m1W, m1b, m2W, m2b, linW, linb,
           do_relu):
    n = xn.shape[0]
    e = src.shape[0]
    pre = _make_pre_call(n)
    a, b, xl, selfmsg = pre(
        xn, m1W[:_D], m1W[_D:], m1b.reshape(1, _D), linW,
        m2W.reshape(1, _D), jnp.broadcast_to(m2b.reshape(1, 1), (1, _D)))
    wparams = jnp.concatenate([m2W[:, 0], jnp.broadcast_to(m2b, (_LANE,))])
    n_pad = -(-n // (8 * _NS)) * (8 * _NS)
    zeros = jnp.zeros((n_pad // _NS, _D), jnp.float32)
    zeros2 = jnp.zeros((n_pad,), jnp.int32)
    ones = jnp.ones((_K,), jnp.int32)
    agg, cnt = _make_edge_call(n, e)(
        a, b, xl, src, dst, wparams, zeros, zeros2, ones)
    if cnt_vec is None:
        cnt2 = cnt.reshape(_NC, n_pad)
        cnt_vec = (cnt2[0] + cnt2[1])[:n].astype(jnp.float32).reshape(n, 1)
    out = _make_combine_call(n, do_relu)(
        agg, cnt_vec, selfmsg, linb.reshape(1, _D))
    return out, cnt_vec


def kernel(x, edge_index, Wl1, bl1, c1_m1W, c1_m1b, c1_m2W, c1_m2b, c1_linW,
           c1_linb, c2_m1W, c2_m1b, c2_m2W, c2_m2b, c2_linW, c2_linb):
    src = edge_index[0]
    dst = edge_index[1]
    x32, cnt_vec = _layer(x, src, dst, None, c1_m1W, c1_m1b, c1_m2W,
                          c1_m2b, c1_linW, c1_linb, do_relu=True)
    h2, _ = _layer(x32, src, dst, cnt_vec, c2_m1W, c2_m1b, c2_m2W,
                   c2_m2b, c2_linW, c2_linb, do_relu=False)
    return (x32, h2)
